# baseline (device time: 134880 ns/iter reference)
import functools

import jax
import jax.numpy as jnp
from jax import lax
from jax.experimental import pallas as pl
from jax.experimental.pallas import tpu as pltpu

N_DEV = 4
N_TOK = 2048
D = 1024
E_LOCAL = 8
N_EXP = 32
CHUNK = N_TOK // N_DEV
C_CAP = 160


def kernel(x, router_W, route_idx, expert_W, shared_W):
    x_bf = x.astype(jnp.bfloat16)
    router_bf = router_W.astype(jnp.bfloat16)
    expert_bf = expert_W.astype(jnp.bfloat16)
    shared_bf = shared_W.astype(jnp.bfloat16)

    def body(
        x_ref,
        router_ref,
        route_ref,
        expert_ref,
        shared_ref,
        out_ref,
        s_buf,
        y_buf,
        stage,
        land,
        ag_own_cw,
        ag_own_ccw,
        ag_land_cw,
        ag_land_ccw,
        rs_send,
        rs_recv,
        ag_send_cw,
        ag_recv_cw,
        ag_send_ccw,
        ag_recv_ccw,
    ):
        my = lax.axis_index("i")
        left = (my - 1) % N_DEV
        right = (my + 1) % N_DEV

        barrier = pltpu.get_barrier_semaphore()
        for nbr in (left, right):
            pl.semaphore_signal(
                barrier, inc=1, device_id=(nbr,),
                device_id_type=pl.DeviceIdType.MESH,
            )
        pl.semaphore_wait(barrier, 2)

        col32 = lax.broadcasted_iota(jnp.int32, (N_TOK, N_EXP), 1)
        scores = jnp.dot(
            x_ref[...], router_ref[...], preferred_element_type=jnp.float32
        )
        s_max = jnp.max(scores, axis=-1, keepdims=True)
        e_un = jnp.exp(scores - s_max)
        probs = e_un / jnp.sum(e_un, axis=-1, keepdims=True)
        routed = col32 == route_ref[...]
        gate = jnp.where(routed, probs, 0.0)
        run = routed.astype(jnp.int32)
        sh = 1
        while sh < N_TOK:
            run = run + jnp.concatenate(
                [jnp.zeros((sh, N_EXP), jnp.int32), run[: N_TOK - sh, :]],
                axis=0,
            )
            sh *= 2
        slot = run - 1
        colC = lax.broadcasted_iota(jnp.int32, (N_TOK, C_CAP), 1)

        for le in range(E_LOCAL):
            e_glob = my * E_LOCAL + le
            m_e = routed & (col32 == e_glob)
            sl_e = jnp.sum(
                jnp.where(m_e, slot, 0), axis=-1, keepdims=True
            )
            v_e = jnp.sum(m_e.astype(jnp.int32), axis=-1, keepdims=True)
            g_e = jnp.sum(
                jnp.where(m_e, gate, 0.0), axis=-1, keepdims=True
            )
            hit = (colC == sl_e) & (v_e > 0)
            s_buf[le] = hit.astype(jnp.bfloat16)
            st_w = jnp.where(hit, g_e, 0.0).astype(jnp.bfloat16)
            x_g = lax.dot_general(
                st_w,
                x_ref[...],
                (((0,), (0,)), ((), ())),
                preferred_element_type=jnp.float32,
            )
            y_buf[le] = jnp.dot(
                x_g.astype(jnp.bfloat16),
                expert_ref[le],
                preferred_element_type=jnp.float32,
            ).astype(jnp.bfloat16)

        def expert_partial(c):
            rows = pl.ds(c * CHUNK, CHUNK)
            acc_c = jnp.zeros((CHUNK, D), jnp.float32)
            for le in range(E_LOCAL):
                acc_c = acc_c + jnp.dot(
                    s_buf[le, rows, :],
                    y_buf[le],
                    preferred_element_type=jnp.float32,
                )
            return acc_c

        def rs_rdma(s):
            return pltpu.make_async_remote_copy(
                src_ref=stage.at[s],
                dst_ref=land.at[s],
                send_sem=rs_send.at[s],
                recv_sem=rs_recv.at[s],
                device_id=(right,),
                device_id_type=pl.DeviceIdType.MESH,
            )

        handles = []
        acc = expert_partial(my % N_DEV)
        stage[0] = acc.astype(jnp.bfloat16)
        r = rs_rdma(0)
        r.start()
        handles.append(r)
        q = (my + 1) % N_DEV
        q_rows = pl.ds(q * CHUNK, CHUNK)
        shared_q = None
        for s in range(1, N_DEV):
            c = (my - s) % N_DEV
            acc = expert_partial(c)
            if s == N_DEV - 1:
                shared_q = jnp.dot(
                    x_ref[q_rows, :], shared_ref[...],
                    preferred_element_type=jnp.float32,
                )
            handles[s - 1].wait_recv()
            merged = acc + land[s - 1].astype(jnp.float32)
            if s < N_DEV - 1:
                stage[s] = merged.astype(jnp.bfloat16)
                r = rs_rdma(s)
                r.start()
                handles.append(r)
        final_q = merged + shared_q
        out_ref[q_rows, :] = final_q
        fin_bf = final_q.astype(jnp.bfloat16)
        ag_own_cw[...] = fin_bf[:, 0 : D // 2]
        ag_own_ccw[...] = fin_bf[:, D // 2 : D]

        H = D // 2
        ag_cw, ag_ccw = [], []
        for s in range(N_DEV - 1):
            if s > 0:
                ag_cw[s - 1].wait_recv()
                ag_ccw[s - 1].wait_recv()
            r_cw = pltpu.make_async_remote_copy(
                src_ref=ag_own_cw if s == 0 else ag_land_cw.at[s - 1],
                dst_ref=ag_land_cw.at[s],
                send_sem=ag_send_cw.at[s],
                recv_sem=ag_recv_cw.at[s],
                device_id=(right,),
                device_id_type=pl.DeviceIdType.MESH,
            )
            r_cw.start()
            ag_cw.append(r_cw)
            r_ccw = pltpu.make_async_remote_copy(
                src_ref=ag_own_ccw if s == 0 else ag_land_ccw.at[s - 1],
                dst_ref=ag_land_ccw.at[s],
                send_sem=ag_send_ccw.at[s],
                recv_sem=ag_recv_ccw.at[s],
                device_id=(left,),
                device_id_type=pl.DeviceIdType.MESH,
            )
            r_ccw.start()
            ag_ccw.append(r_ccw)
            if s > 0:
                c_cw = (my - (s - 1)) % N_DEV
                out_ref[pl.ds(c_cw * CHUNK, CHUNK), 0:H] = ag_land_cw[
                    s - 1
                ].astype(jnp.float32)
                c_ccw = (my + 2 + (s - 1)) % N_DEV
                out_ref[pl.ds(c_ccw * CHUNK, CHUNK), H:D] = ag_land_ccw[
                    s - 1
                ].astype(jnp.float32)
        last = N_DEV - 2
        ag_cw[last].wait_recv()
        ag_ccw[last].wait_recv()
        c_cw = (my - last) % N_DEV
        out_ref[pl.ds(c_cw * CHUNK, CHUNK), 0:H] = ag_land_cw[last].astype(
            jnp.float32
        )
        c_ccw = (my + 2 + last) % N_DEV
        out_ref[pl.ds(c_ccw * CHUNK, CHUNK), H:D] = ag_land_ccw[last].astype(
            jnp.float32
        )
        handles.extend(ag_cw)
        handles.extend(ag_ccw)

        for r in handles:
            r.wait_send()

        @functools.partial(pl.run_scoped, second=pltpu.SemaphoreType.REGULAR)
        def _(second):
            for nbr in (left, right):
                pl.semaphore_signal(
                    second, inc=1, device_id=(nbr,),
                    device_id_type=pl.DeviceIdType.MESH,
                )
            pl.semaphore_wait(second, 2)

    return pl.pallas_call(
        body,
        out_shape=jax.ShapeDtypeStruct((N_TOK, D), jnp.float32),
        in_specs=[pl.BlockSpec(memory_space=pltpu.VMEM)] * 5,
        out_specs=pl.BlockSpec(memory_space=pltpu.VMEM),
        scratch_shapes=[
            pltpu.VMEM((E_LOCAL, N_TOK, C_CAP), jnp.bfloat16),
            pltpu.VMEM((E_LOCAL, C_CAP, D), jnp.bfloat16),
            pltpu.VMEM((N_DEV - 1, CHUNK, D), jnp.bfloat16),
            pltpu.VMEM((N_DEV - 1, CHUNK, D), jnp.bfloat16),
            pltpu.VMEM((CHUNK, D // 2), jnp.bfloat16),
            pltpu.VMEM((CHUNK, D // 2), jnp.bfloat16),
            pltpu.VMEM((N_DEV - 1, CHUNK, D // 2), jnp.bfloat16),
            pltpu.VMEM((N_DEV - 1, CHUNK, D // 2), jnp.bfloat16),
            pltpu.SemaphoreType.DMA((N_DEV - 1,)),
            pltpu.SemaphoreType.DMA((N_DEV - 1,)),
            pltpu.SemaphoreType.DMA((N_DEV - 1,)),
            pltpu.SemaphoreType.DMA((N_DEV - 1,)),
            pltpu.SemaphoreType.DMA((N_DEV - 1,)),
            pltpu.SemaphoreType.DMA((N_DEV - 1,)),
        ],
        compiler_params=pltpu.CompilerParams(
            collective_id=0, vmem_limit_bytes=100 * 1024 * 1024
        ),
    )(x_bf, router_bf, route_idx, expert_bf, shared_bf)


# device time: 118558 ns/iter; 1.1377x vs baseline; 1.1377x over previous
import functools

import jax
import jax.numpy as jnp
from jax import lax
from jax.experimental import pallas as pl
from jax.experimental.pallas import tpu as pltpu

N_DEV = 4
N_TOK = 2048
D = 1024
E_LOCAL = 8
N_EXP = 32
CHUNK = N_TOK // N_DEV
H = D // 2
C_CAP = 128
SLOTS = E_LOCAL * C_CAP


def kernel(x, router_W, route_idx, expert_W, shared_W):
    x_bf = x.astype(jnp.bfloat16)
    router_bf = router_W.astype(jnp.bfloat16)
    expert_bf = expert_W.astype(jnp.bfloat16)
    shared_bf = shared_W.astype(jnp.bfloat16)

    def body(
        x_ref,
        router_ref,
        route_ref,
        expert_ref,
        shared_ref,
        out_ref,
        s_buf,
        y_buf,
        stage_cw,
        land_cw,
        stage_ccw,
        land_ccw,
        ag_own_cw,
        ag_own_ccw,
        ag_land_cw,
        ag_land_ccw,
        cw_send,
        cw_recv,
        ccw_send,
        ccw_recv,
        ag_send_cw,
        ag_recv_cw,
        ag_send_ccw,
        ag_recv_ccw,
    ):
        my = lax.axis_index("i")
        left = (my - 1) % N_DEV
        right = (my + 1) % N_DEV

        barrier = pltpu.get_barrier_semaphore()
        for nbr in (left, right):
            pl.semaphore_signal(
                barrier, inc=1, device_id=(nbr,),
                device_id_type=pl.DeviceIdType.MESH,
            )
        pl.semaphore_wait(barrier, 2)

        col32 = lax.broadcasted_iota(jnp.int32, (N_TOK, N_EXP), 1)
        scores = jnp.dot(
            x_ref[...], router_ref[...], preferred_element_type=jnp.float32
        )
        s_max = jnp.max(scores, axis=-1, keepdims=True)
        e_un = jnp.exp(scores - s_max)
        probs = e_un / jnp.sum(e_un, axis=-1, keepdims=True)
        routed = col32 == route_ref[...]
        gate = jnp.where(routed, probs, 0.0)
        run = routed.astype(jnp.int32)
        sh = 1
        while sh < N_TOK:
            run = run + jnp.concatenate(
                [jnp.zeros((sh, N_EXP), jnp.int32), run[: N_TOK - sh, :]],
                axis=0,
            )
            sh *= 2
        slot = run - 1
        colC = lax.broadcasted_iota(jnp.int32, (N_TOK, C_CAP), 1)

        g_cols = []
        for le in range(E_LOCAL):
            e_glob = my * E_LOCAL + le
            m_e = routed & (col32 == e_glob)
            sl_e = jnp.sum(
                jnp.where(m_e, slot, 0), axis=-1, keepdims=True
            )
            v_e = jnp.sum(m_e.astype(jnp.int32), axis=-1, keepdims=True)
            g_cols.append(
                jnp.sum(jnp.where(m_e, gate, 0.0), axis=-1, keepdims=True)
            )
            hit = (colC == sl_e) & (v_e > 0)
            s_buf[:, le * C_CAP : (le + 1) * C_CAP] = hit.astype(jnp.bfloat16)

        S = s_buf[...]
        x_g = lax.dot_general(
            S, x_ref[...], (((0,), (0,)), ((), ())),
            preferred_element_type=jnp.float32,
        )
        G = jnp.concatenate(g_cols, axis=1).astype(jnp.bfloat16)
        gs = lax.dot_general(
            S, G, (((0,), (0,)), ((), ())),
            preferred_element_type=jnp.float32,
        )
        brow = lax.broadcasted_iota(jnp.int32, (SLOTS, E_LOCAL), 0) // C_CAP
        bcol = lax.broadcasted_iota(jnp.int32, (SLOTS, E_LOCAL), 1)
        g_slot = jnp.sum(
            jnp.where(brow == bcol, gs, 0.0), axis=-1, keepdims=True
        )
        for le in range(E_LOCAL):
            blk = slice(le * C_CAP, (le + 1) * C_CAP)
            xg_le = (g_slot[blk, :] * x_g[blk, :]).astype(jnp.bfloat16)
            y_buf[blk, :] = jnp.dot(
                xg_le, expert_ref[le], preferred_element_type=jnp.float32
            ).astype(jnp.bfloat16)

        def partial(c):
            return jnp.dot(
                s_buf[pl.ds(c * CHUNK, CHUNK), :],
                y_buf[...],
                preferred_element_type=jnp.float32,
            )

        handles = []

        def mk(src, dst, ssem, rsem, dev):
            r = pltpu.make_async_remote_copy(
                src_ref=src, dst_ref=dst, send_sem=ssem, recv_sem=rsem,
                device_id=(dev,), device_id_type=pl.DeviceIdType.MESH,
            )
            r.start()
            handles.append(r)
            return r

        c0 = my % N_DEV
        acc0 = partial(c0)
        out_ref[pl.ds(c0 * CHUNK, CHUNK), :] = acc0
        stage_cw[0] = acc0[:, 0:H].astype(jnp.bfloat16)
        cw_h = [mk(stage_cw.at[0], land_cw.at[0], cw_send.at[0],
                   cw_recv.at[0], right)]
        stage_ccw[0] = acc0[:, H:D].astype(jnp.bfloat16)
        ccw_h = [mk(stage_ccw.at[0], land_ccw.at[0], ccw_send.at[0],
                    ccw_recv.at[0], left)]

        c1 = (my + 1) % N_DEV
        acc1 = partial(c1)
        out_ref[pl.ds(c1 * CHUNK, CHUNK), :] = acc1
        ccw_h[0].wait_recv()
        m_ccw = acc1[:, H:D] + land_ccw[0].astype(jnp.float32)
        stage_ccw[1] = m_ccw.astype(jnp.bfloat16)
        ccw_h.append(mk(stage_ccw.at[1], land_ccw.at[1], ccw_send.at[1],
                        ccw_recv.at[1], left))

        c2 = (my + 3) % N_DEV
        acc2 = partial(c2)
        out_ref[pl.ds(c2 * CHUNK, CHUNK), :] = acc2
        cw_h[0].wait_recv()
        m_cw = acc2[:, 0:H] + land_cw[0].astype(jnp.float32)
        stage_cw[1] = m_cw.astype(jnp.bfloat16)
        cw_h.append(mk(stage_cw.at[1], land_cw.at[1], cw_send.at[1],
                       cw_recv.at[1], right))

        c3 = (my + 2) % N_DEV
        acc3 = partial(c3)
        cw_h[1].wait_recv()
        m_cw2 = acc3[:, 0:H] + land_cw[1].astype(jnp.float32)
        stage_cw[2] = m_cw2.astype(jnp.bfloat16)
        cw_h.append(mk(stage_cw.at[2], land_cw.at[2], cw_send.at[2],
                       cw_recv.at[2], right))
        ccw_h[1].wait_recv()
        m_ccw2 = acc3[:, H:D] + land_ccw[1].astype(jnp.float32)
        stage_ccw[2] = m_ccw2.astype(jnp.bfloat16)
        ccw_h.append(mk(stage_ccw.at[2], land_ccw.at[2], ccw_send.at[2],
                        ccw_recv.at[2], left))

        sh_cw = jnp.dot(
            x_ref[pl.ds(c1 * CHUNK, CHUNK), :], shared_ref[:, 0:H],
            preferred_element_type=jnp.float32,
        )
        sh_ccw = jnp.dot(
            x_ref[pl.ds(c2 * CHUNK, CHUNK), :], shared_ref[:, H:D],
            preferred_element_type=jnp.float32,
        )

        cw_h[2].wait_recv()
        final_cw = out_ref[pl.ds(c1 * CHUNK, CHUNK), 0:H] + land_cw[2].astype(
            jnp.float32
        ) + sh_cw
        out_ref[pl.ds(c1 * CHUNK, CHUNK), 0:H] = final_cw
        ag_own_cw[...] = final_cw.astype(jnp.bfloat16)
        ccw_h[2].wait_recv()
        final_ccw = out_ref[pl.ds(c2 * CHUNK, CHUNK), H:D] + land_ccw[
            2
        ].astype(jnp.float32) + sh_ccw
        out_ref[pl.ds(c2 * CHUNK, CHUNK), H:D] = final_ccw
        ag_own_ccw[...] = final_ccw.astype(jnp.bfloat16)

        ag_cw, ag_ccw = [], []
        for s in range(N_DEV - 1):
            if s > 0:
                ag_cw[s - 1].wait_recv()
                ag_ccw[s - 1].wait_recv()
            ag_cw.append(
                mk(ag_own_cw if s == 0 else ag_land_cw.at[s - 1],
                   ag_land_cw.at[s], ag_send_cw.at[s], ag_recv_cw.at[s],
                   right)
            )
            ag_ccw.append(
                mk(ag_own_ccw if s == 0 else ag_land_ccw.at[s - 1],
                   ag_land_ccw.at[s], ag_send_ccw.at[s], ag_recv_ccw.at[s],
                   left)
            )
            if s > 0:
                c_cw = (my - (s - 1)) % N_DEV
                out_ref[pl.ds(c_cw * CHUNK, CHUNK), 0:H] = ag_land_cw[
                    s - 1
                ].astype(jnp.float32)
                c_ccw = (my + (s - 1)) % N_DEV
                out_ref[pl.ds(c_ccw * CHUNK, CHUNK), H:D] = ag_land_ccw[
                    s - 1
                ].astype(jnp.float32)
        last = N_DEV - 2
        ag_cw[last].wait_recv()
        ag_ccw[last].wait_recv()
        c_cw = (my - last) % N_DEV
        out_ref[pl.ds(c_cw * CHUNK, CHUNK), 0:H] = ag_land_cw[last].astype(
            jnp.float32
        )
        c_ccw = (my + last) % N_DEV
        out_ref[pl.ds(c_ccw * CHUNK, CHUNK), H:D] = ag_land_ccw[last].astype(
            jnp.float32
        )

        for r in handles:
            r.wait_send()

        @functools.partial(pl.run_scoped, second=pltpu.SemaphoreType.REGULAR)
        def _(second):
            for nbr in (left, right):
                pl.semaphore_signal(
                    second, inc=1, device_id=(nbr,),
                    device_id_type=pl.DeviceIdType.MESH,
                )
            pl.semaphore_wait(second, 2)

    return pl.pallas_call(
        body,
        out_shape=jax.ShapeDtypeStruct((N_TOK, D), jnp.float32),
        in_specs=[pl.BlockSpec(memory_space=pltpu.VMEM)] * 5,
        out_specs=pl.BlockSpec(memory_space=pltpu.VMEM),
        scratch_shapes=[
            pltpu.VMEM((N_TOK, SLOTS), jnp.bfloat16),
            pltpu.VMEM((SLOTS, D), jnp.bfloat16),
            pltpu.VMEM((N_DEV - 1, CHUNK, H), jnp.bfloat16),
            pltpu.VMEM((N_DEV - 1, CHUNK, H), jnp.bfloat16),
            pltpu.VMEM((N_DEV - 1, CHUNK, H), jnp.bfloat16),
            pltpu.VMEM((N_DEV - 1, CHUNK, H), jnp.bfloat16),
            pltpu.VMEM((CHUNK, H), jnp.bfloat16),
            pltpu.VMEM((CHUNK, H), jnp.bfloat16),
            pltpu.VMEM((N_DEV - 1, CHUNK, H), jnp.bfloat16),
            pltpu.VMEM((N_DEV - 1, CHUNK, H), jnp.bfloat16),
        ] + [pltpu.SemaphoreType.DMA((N_DEV - 1,))] * 8,
        compiler_params=pltpu.CompilerParams(
            collective_id=0, vmem_limit_bytes=100 * 1024 * 1024
        ),
    )(x_bf, router_bf, route_idx, expert_bf, shared_bf)


# device time: 95391 ns/iter; 1.4140x vs baseline; 1.2429x over previous
import functools

import jax
import jax.numpy as jnp
from jax import lax
from jax.experimental import pallas as pl
from jax.experimental.pallas import tpu as pltpu

N_DEV = 4
N_TOK = 2048
D = 1024
E_LOCAL = 8
N_EXP = 32
CHUNK = N_TOK // N_DEV
H = D // 2
C_CAP = 128
SLOTS = E_LOCAL * C_CAP


def kernel(x, router_W, route_idx, expert_W, shared_W):
    def body(
        x_ref,
        router_ref,
        route_ref,
        expert_hbm,
        shared_ref,
        out_ref,
        xbf,
        s_buf,
        y_buf,
        ew,
        stage_cw,
        land_cw,
        stage_ccw,
        land_ccw,
        ew_sem,
        cw_send,
        cw_recv,
        ccw_send,
        ccw_recv,
        ag_send_cw,
        ag_recv_cw,
        ag_send_ccw,
        ag_recv_ccw,
    ):
        my = lax.axis_index("i")
        left = (my - 1) % N_DEV
        right = (my + 1) % N_DEV

        def ew_copy(le):
            return pltpu.make_async_copy(
                expert_hbm.at[le], ew.at[le % 2], ew_sem.at[le % 2]
            )

        ew_copy(0).start()

        barrier = pltpu.get_barrier_semaphore()
        for nbr in (left, right):
            pl.semaphore_signal(
                barrier, inc=1, device_id=(nbr,),
                device_id_type=pl.DeviceIdType.MESH,
            )
        pl.semaphore_wait(barrier, 2)

        xbf[...] = x_ref[...].astype(jnp.bfloat16)

        col32 = lax.broadcasted_iota(jnp.int32, (N_TOK, N_EXP), 1)
        scores = jnp.dot(
            xbf[...],
            router_ref[...].astype(jnp.bfloat16),
            preferred_element_type=jnp.float32,
        )
        s_max = jnp.max(scores, axis=-1, keepdims=True)
        e_un = jnp.exp(scores - s_max)
        probs = e_un / jnp.sum(e_un, axis=-1, keepdims=True)
        routed = col32 == route_ref[...]
        gate = jnp.where(routed, probs, 0.0)
        run = routed.astype(jnp.int32)
        sh = 1
        while sh < N_TOK:
            run = run + jnp.concatenate(
                [jnp.zeros((sh, N_EXP), jnp.int32), run[: N_TOK - sh, :]],
                axis=0,
            )
            sh *= 2
        slot = run - 1
        colC = lax.broadcasted_iota(jnp.int32, (N_TOK, C_CAP), 1)

        g_cols = []
        for le in range(E_LOCAL):
            e_glob = my * E_LOCAL + le
            m_e = routed & (col32 == e_glob)
            sl_e = jnp.sum(
                jnp.where(m_e, slot, 0), axis=-1, keepdims=True
            )
            v_e = jnp.sum(m_e.astype(jnp.int32), axis=-1, keepdims=True)
            g_cols.append(
                jnp.sum(jnp.where(m_e, gate, 0.0), axis=-1, keepdims=True)
            )
            hit = (colC == sl_e) & (v_e > 0)
            s_buf[:, le * C_CAP : (le + 1) * C_CAP] = hit.astype(jnp.bfloat16)

        S = s_buf[...]
        x_g = lax.dot_general(
            S, xbf[...], (((0,), (0,)), ((), ())),
            preferred_element_type=jnp.float32,
        )
        G = jnp.concatenate(g_cols, axis=1).astype(jnp.bfloat16)
        gs = lax.dot_general(
            S, G, (((0,), (0,)), ((), ())),
            preferred_element_type=jnp.float32,
        )
        brow = lax.broadcasted_iota(jnp.int32, (SLOTS, E_LOCAL), 0) // C_CAP
        bcol = lax.broadcasted_iota(jnp.int32, (SLOTS, E_LOCAL), 1)
        g_slot = jnp.sum(
            jnp.where(brow == bcol, gs, 0.0), axis=-1, keepdims=True
        )
        for le in range(E_LOCAL):
            if le + 1 < E_LOCAL:
                ew_copy(le + 1).start()
            ew_copy(le).wait()
            w_bf = ew[le % 2].astype(jnp.bfloat16)
            blk = slice(le * C_CAP, (le + 1) * C_CAP)
            xg_le = (g_slot[blk, :] * x_g[blk, :]).astype(jnp.bfloat16)
            y_buf[blk, :] = jnp.dot(
                xg_le, w_bf, preferred_element_type=jnp.float32
            ).astype(jnp.bfloat16)

        def partial(c):
            return jnp.dot(
                s_buf[pl.ds(c * CHUNK, CHUNK), :],
                y_buf[...],
                preferred_element_type=jnp.float32,
            )

        handles = []

        def mk(src, dst, ssem, rsem, dev):
            r = pltpu.make_async_remote_copy(
                src_ref=src, dst_ref=dst, send_sem=ssem, recv_sem=rsem,
                device_id=(dev,), device_id_type=pl.DeviceIdType.MESH,
            )
            r.start()
            handles.append(r)
            return r

        c0 = my % N_DEV
        acc0 = partial(c0)
        stage_cw[0] = acc0[:, 0:H].astype(jnp.bfloat16)
        cw_h = [mk(stage_cw.at[0], land_cw.at[0], cw_send.at[0],
                   cw_recv.at[0], right)]
        stage_ccw[0] = acc0[:, H:D].astype(jnp.bfloat16)
        ccw_h = [mk(stage_ccw.at[0], land_ccw.at[0], ccw_send.at[0],
                    ccw_recv.at[0], left)]

        c1 = (my + 1) % N_DEV
        acc1 = partial(c1)
        out_ref[pl.ds(c1 * CHUNK, CHUNK), 0:H] = acc1[:, 0:H].astype(
            jnp.bfloat16
        )
        ccw_h[0].wait_recv()
        m_ccw = acc1[:, H:D] + land_ccw[0].astype(jnp.float32)
        stage_ccw[1] = m_ccw.astype(jnp.bfloat16)
        ccw_h.append(mk(stage_ccw.at[1], land_ccw.at[1], ccw_send.at[1],
                        ccw_recv.at[1], left))

        c2 = (my + 3) % N_DEV
        acc2 = partial(c2)
        out_ref[pl.ds(c2 * CHUNK, CHUNK), H:D] = acc2[:, H:D].astype(
            jnp.bfloat16
        )
        cw_h[0].wait_recv()
        m_cw = acc2[:, 0:H] + land_cw[0].astype(jnp.float32)
        stage_cw[1] = m_cw.astype(jnp.bfloat16)
        cw_h.append(mk(stage_cw.at[1], land_cw.at[1], cw_send.at[1],
                       cw_recv.at[1], right))

        c3 = (my + 2) % N_DEV
        acc3 = partial(c3)
        cw_h[1].wait_recv()
        m_cw2 = acc3[:, 0:H] + land_cw[1].astype(jnp.float32)
        stage_cw[2] = m_cw2.astype(jnp.bfloat16)
        cw_h.append(mk(stage_cw.at[2], land_cw.at[2], cw_send.at[2],
                       cw_recv.at[2], right))
        ccw_h[1].wait_recv()
        m_ccw2 = acc3[:, H:D] + land_ccw[1].astype(jnp.float32)
        stage_ccw[2] = m_ccw2.astype(jnp.bfloat16)
        ccw_h.append(mk(stage_ccw.at[2], land_ccw.at[2], ccw_send.at[2],
                        ccw_recv.at[2], left))

        sh_cw = jnp.dot(
            xbf[pl.ds(c1 * CHUNK, CHUNK), :],
            shared_ref[:, 0:H].astype(jnp.bfloat16),
            preferred_element_type=jnp.float32,
        )
        sh_ccw = jnp.dot(
            xbf[pl.ds(c2 * CHUNK, CHUNK), :],
            shared_ref[:, H:D].astype(jnp.bfloat16),
            preferred_element_type=jnp.float32,
        )

        cw_h[2].wait_recv()
        final_cw = (
            out_ref[pl.ds(c1 * CHUNK, CHUNK), 0:H].astype(jnp.float32)
            + land_cw[2].astype(jnp.float32)
            + sh_cw
        )
        out_ref[pl.ds(c1 * CHUNK, CHUNK), 0:H] = final_cw.astype(jnp.bfloat16)
        ccw_h[2].wait_recv()
        final_ccw = (
            out_ref[pl.ds(c2 * CHUNK, CHUNK), H:D].astype(jnp.float32)
            + land_ccw[2].astype(jnp.float32)
            + sh_ccw
        )
        out_ref[pl.ds(c2 * CHUNK, CHUNK), H:D] = final_ccw.astype(jnp.bfloat16)

        ag_cw, ag_ccw = [], []
        for s in range(N_DEV - 1):
            if s > 0:
                ag_cw[s - 1].wait_recv()
                ag_ccw[s - 1].wait_recv()
            rc = ((my + 1 - s) % N_DEV) * CHUNK
            ag_cw.append(
                mk(out_ref.at[pl.ds(rc, CHUNK), pl.ds(0, H)],
                   out_ref.at[pl.ds(rc, CHUNK), pl.ds(0, H)],
                   ag_send_cw.at[s], ag_recv_cw.at[s], right)
            )
            rcc = ((my - 1 + s) % N_DEV) * CHUNK
            ag_ccw.append(
                mk(out_ref.at[pl.ds(rcc, CHUNK), pl.ds(H, H)],
                   out_ref.at[pl.ds(rcc, CHUNK), pl.ds(H, H)],
                   ag_send_ccw.at[s], ag_recv_ccw.at[s], left)
            )
        ag_cw[N_DEV - 2].wait_recv()
        ag_ccw[N_DEV - 2].wait_recv()

        for r in handles:
            r.wait_send()

        @functools.partial(pl.run_scoped, second=pltpu.SemaphoreType.REGULAR)
        def _(second):
            for nbr in (left, right):
                pl.semaphore_signal(
                    second, inc=1, device_id=(nbr,),
                    device_id_type=pl.DeviceIdType.MESH,
                )
            pl.semaphore_wait(second, 2)

    return pl.pallas_call(
        body,
        out_shape=jax.ShapeDtypeStruct((N_TOK, D), jnp.bfloat16),
        in_specs=[
            pl.BlockSpec(memory_space=pltpu.VMEM),
            pl.BlockSpec(memory_space=pltpu.VMEM),
            pl.BlockSpec(memory_space=pltpu.VMEM),
            pl.BlockSpec(memory_space=pltpu.MemorySpace.HBM),
            pl.BlockSpec(memory_space=pltpu.VMEM),
        ],
        out_specs=pl.BlockSpec(memory_space=pltpu.VMEM),
        scratch_shapes=[
            pltpu.VMEM((N_TOK, D), jnp.bfloat16),
            pltpu.VMEM((N_TOK, SLOTS), jnp.bfloat16),
            pltpu.VMEM((SLOTS, D), jnp.bfloat16),
            pltpu.VMEM((2, D, D), jnp.float32),
            pltpu.VMEM((N_DEV - 1, CHUNK, H), jnp.bfloat16),
            pltpu.VMEM((N_DEV - 1, CHUNK, H), jnp.bfloat16),
            pltpu.VMEM((N_DEV - 1, CHUNK, H), jnp.bfloat16),
            pltpu.VMEM((N_DEV - 1, CHUNK, H), jnp.bfloat16),
            pltpu.SemaphoreType.DMA((2,)),
        ] + [pltpu.SemaphoreType.DMA((N_DEV - 1,))] * 8,
        compiler_params=pltpu.CompilerParams(
            collective_id=0, vmem_limit_bytes=100 * 1024 * 1024
        ),
    )(x, router_W, route_idx, expert_W, shared_W)
